# Initial kernel scaffold; baseline (speedup 1.0000x reference)
import jax
import jax.numpy as jnp
from jax.experimental import pallas as pl


def kernel(x, W1, g1, b1, W2, g2, b2, W3, g3, b3, W4, g4, b4, W5, g5, b5, Wl1, g6, b6, Wl2, bl2, g7, b7, Wl3, bl3):
    raise NotImplementedError("write your pallas kernel here")


# speedup vs baseline: 1.2470x; 1.2470x over previous
import functools

import jax
import jax.numpy as jnp
from jax.experimental import pallas as pl

KNN = 20
EPS = 1e-5
BLK = 256
_HI = jax.lax.Precision.HIGHEST


def _sel_body(xT_ref, xx_ref, xxb_ref, xi_ref, w_ref, mx_ref, h_ref):
    xT = xT_ref[0]
    xxj = xx_ref[0]
    xxb = xxb_ref[0]
    xi = xi_ref[0]
    W = w_ref[...]
    N = xT.shape[0]
    O = W.shape[0]

    ir = jax.lax.broadcasted_iota(jnp.int32, (BLK, BLK), 0)
    ic = jax.lax.broadcasted_iota(jnp.int32, (BLK, BLK), 1)
    ident = (ir == ic).astype(jnp.float32)
    xxi = jax.lax.dot_general(ident, xxb, (((1,), (1,)), ((), ())),
                              preferred_element_type=jnp.float32,
                              precision=_HI)
    inner = jax.lax.dot_general(xi, xT, (((1,), (1,)), ((), ())),
                                preferred_element_type=jnp.float32)
    D = (-xxj) - (-2.0 * inner)
    D = D - xxi
    iota = jax.lax.broadcasted_iota(jnp.int32, (BLK, N), 1)

    mx = jnp.full((BLK, O), -jnp.inf, jnp.float32)
    for t in range(KNN):
        m = jnp.max(D, axis=1, keepdims=True)
        cidx = jnp.min(jnp.where(D == m, iota, N), axis=1, keepdims=True)
        onehot = iota == cidx
        xg = jax.lax.dot_general(
            onehot.astype(jnp.float32), xT, (((1,), (0,)), ((), ())),
            preferred_element_type=jnp.float32, precision=_HI)
        feat = jnp.concatenate([xg - xi, xi], axis=1)
        if O == 256:
            hT = jax.lax.dot_general(W, feat, (((1,), (1,)), ((), ())),
                                     preferred_element_type=jnp.float32)
            h = jax.lax.dot_general(ident, hT, (((1,), (1,)), ((), ())),
                                    preferred_element_type=jnp.float32,
                                    precision=_HI)
        else:
            h = jax.lax.dot_general(feat, W, (((1,), (1,)), ((), ())),
                                    preferred_element_type=jnp.float32)
            hT = jax.lax.dot_general(h, ident, (((0,), (0,)), ((), ())),
                                     preferred_element_type=jnp.float32,
                                     precision=_HI)
        mx = jnp.maximum(mx, h)
        h_ref[0, t] = hT
        D = jnp.where(onehot, -jnp.inf, D)

    mx_ref[0] = mx


def _bn_body(mx_ref, m_ref, v_ref, g_ref, b_ref, out_ref):
    den = jnp.sqrt(v_ref[...] + EPS)
    h = (mx_ref[0] - m_ref[...]) / den * g_ref[...] + b_ref[...]
    out_ref[0] = jnp.maximum(h, 0.0)


def _conv5_body(x1_ref, x2_ref, x3_ref, x4_ref, w1_ref, w2_ref, w3_ref,
                w4_ref, h_ref):
    h = jnp.dot(x1_ref[0], w1_ref[...], preferred_element_type=jnp.float32)
    h = h + jnp.dot(x2_ref[0], w2_ref[...],
                    preferred_element_type=jnp.float32)
    h = h + jnp.dot(x3_ref[0], w3_ref[...],
                    preferred_element_type=jnp.float32)
    h = h + jnp.dot(x4_ref[0], w4_ref[...],
                    preferred_element_type=jnp.float32)
    h_ref[0] = h


def _mm_body(a_ref, b_ref, o_ref):
    o_ref[...] = jnp.dot(a_ref[...], b_ref[...],
                         preferred_element_type=jnp.float32)


def _mm(a, b):
    M, K = a.shape
    K2, Np = b.shape
    return pl.pallas_call(
        _mm_body,
        in_specs=[_full((M, K)), _full((K2, Np))],
        out_specs=_full((M, Np)),
        out_shape=jax.ShapeDtypeStruct((M, Np), jnp.float32),
    )(a, b)


def _bn_ref(x, g, b, axes, shape):
    m = jnp.mean(x, axis=axes, keepdims=True)
    v = jnp.mean((x - m) ** 2, axis=axes, keepdims=True)
    return (x - m) / jnp.sqrt(v + EPS) * g.reshape(shape) + b.reshape(shape)


def _leaky(x):
    return jnp.where(x >= 0, x, 0.2 * x)


def _full(shape):
    return pl.BlockSpec(shape, lambda *_: tuple(0 for _ in shape))


def _edgeconv(xT, xC, W, g, b):
    B, N, C = xT.shape
    O = W.shape[0]
    NB = N // BLK
    xx = jnp.sum(xC ** 2, axis=1, keepdims=True)
    mx, h_all = pl.pallas_call(
        _sel_body,
        grid=(B, NB),
        in_specs=[pl.BlockSpec((1, N, C), lambda i, j: (i, 0, 0)),
                  pl.BlockSpec((1, 1, N), lambda i, j: (i, 0, 0)),
                  pl.BlockSpec((1, 1, BLK), lambda i, j: (i, 0, j)),
                  pl.BlockSpec((1, BLK, C), lambda i, j: (i, j, 0)),
                  _full((O, 2 * C))],
        out_specs=[pl.BlockSpec((1, BLK, O), lambda i, j: (i, j, 0)),
                   pl.BlockSpec((1, KNN, O, BLK),
                                lambda i, j: (i, 0, 0, j))],
        out_shape=[jax.ShapeDtypeStruct((B, N, O), jnp.float32),
                   jax.ShapeDtypeStruct((B, KNN, O, N), jnp.float32)],
    )(xT, xx, xx, xT, W)

    hr = jax.lax.optimization_barrier(jnp.transpose(h_all, (0, 2, 3, 1)))
    m = jnp.mean(hr, axis=(0, 2, 3), keepdims=True)
    v = jnp.mean((hr - m) ** 2, axis=(0, 2, 3), keepdims=True)

    out = pl.pallas_call(
        _bn_body,
        grid=(B,),
        in_specs=[pl.BlockSpec((1, N, O), lambda i: (i, 0, 0)),
                  _full((1, O)), _full((1, O)),
                  _full((1, O)), _full((1, O))],
        out_specs=pl.BlockSpec((1, N, O), lambda i: (i, 0, 0)),
        out_shape=jax.ShapeDtypeStruct((B, N, O), jnp.float32),
    )(mx, m.reshape(1, O), v.reshape(1, O), g.reshape(1, O),
      b.reshape(1, O))
    return out


def kernel(x, W1, g1, b1, W2, g2, b2, W3, g3, b3, W4, g4, b4, W5, g5, b5,
           Wl1, g6, b6, Wl2, bl2, g7, b7, Wl3, bl3):
    B, _, N = x.shape
    xT = jnp.swapaxes(x, 1, 2)
    x1 = _edgeconv(xT, x, W1, g1, b1)
    x2 = _edgeconv(x1, jnp.swapaxes(x1, 1, 2), W2, g2, b2)
    x3 = _edgeconv(x2, jnp.swapaxes(x2, 1, 2), W3, g3, b3)
    x4 = _edgeconv(x3, jnp.swapaxes(x3, 1, 2), W4, g4, b4)

    w5T = jnp.transpose(W5)
    ws = [w5T[0:64], w5T[64:128], w5T[128:256], w5T[256:512]]
    h5 = pl.pallas_call(
        _conv5_body,
        grid=(B,),
        in_specs=[pl.BlockSpec((1, N, 64), lambda i: (i, 0, 0)),
                  pl.BlockSpec((1, N, 64), lambda i: (i, 0, 0)),
                  pl.BlockSpec((1, N, 128), lambda i: (i, 0, 0)),
                  pl.BlockSpec((1, N, 256), lambda i: (i, 0, 0)),
                  _full((64, 1024)), _full((64, 1024)),
                  _full((128, 1024)), _full((256, 1024))],
        out_specs=pl.BlockSpec((1, N, 1024), lambda i: (i, 0, 0)),
        out_shape=jax.ShapeDtypeStruct((B, N, 1024), jnp.float32),
    )(x1, x2, x3, x4, *ws)

    h5r = jax.lax.optimization_barrier(jnp.transpose(h5, (0, 2, 1)))
    hb = jax.nn.relu(_bn_ref(h5r, g5, b5, (0, 2), (1, -1, 1)))
    hm = jnp.max(hb, axis=-1)
    ha = jnp.mean(hb, axis=-1)
    hcat = jnp.concatenate((hm, ha), axis=1)

    h = _leaky(_bn_ref(_mm(hcat, Wl1), g6, b6, (0,), (1, -1)))
    h = _leaky(_bn_ref(_mm(h, Wl2) + bl2, g7, b7, (0,), (1, -1)))
    return _mm(h, Wl3) + bl3
